# baseline (speedup 1.0000x reference)
import functools

import jax
import jax.numpy as jnp
from jax import lax
from jax.experimental import pallas as pl
from jax.experimental.pallas import tpu as pltpu
from jax.experimental.pallas import tpu_sc as plsc

_NC, _NS, _L = 2, 16, 16
_NW = _NC * _NS
_NP = 10240
_BSZ = 128


def _deg_partials(dst, np_pad, npt):
    rpt = np_pad // _NS
    nring = 4
    mesh = plsc.VectorSubcoreMesh(core_axis_name="c", subcore_axis_name="s")

    @functools.partial(
        pl.kernel,
        out_type=jax.ShapeDtypeStruct((_NC * np_pad,), jnp.float32),
        mesh=mesh,
        scratch_types=[
            [pltpu.VMEM((_BSZ,), jnp.int32) for _ in range(nring)],
            [pltpu.SemaphoreType.DMA for _ in range(nring)],
            pltpu.VMEM((_BSZ,), jnp.float32),
            pltpu.VMEM((rpt,), jnp.float32),
            pltpu.VMEM_SHARED((np_pad,), jnp.float32),
        ],
    )
    def k(dst_hbm, out_hbm, didx, isem, ones_v, zeros_v, acc):
        c = lax.axis_index("c")
        s = lax.axis_index("s")
        wid = c * _NS + s
        base = wid * npt * _BSZ
        for b in range(nring):
            pltpu.async_copy(dst_hbm.at[pl.ds(base + b * _BSZ, _BSZ)],
                             didx[b], isem[b])
        for j in range(_BSZ // _L):
            ones_v[pl.ds(j * _L, _L)] = jnp.full((_L,), 1.0, jnp.float32)
        for j in range(rpt // _L):
            zeros_v[pl.ds(j * _L, _L)] = jnp.zeros((_L,), jnp.float32)
        pltpu.sync_copy(zeros_v, acc.at[pl.ds(s * rpt, rpt)])
        plsc.subcore_barrier()

        def body(g, carry):
            for u in range(nring):
                j = g * nring + u
                pltpu.make_async_copy(dst_hbm.at[pl.ds(base, _BSZ)],
                                      didx[u], isem[u]).wait()
                pltpu.sync_copy(ones_v, acc.at[didx[u]], add=True)
                pltpu.async_copy(
                    dst_hbm.at[pl.ds(base + (j + nring) * _BSZ, _BSZ)],
                    didx[u], isem[u])
            return carry

        lax.fori_loop(0, npt // nring, body, 0)
        for b in range(nring):
            pltpu.make_async_copy(dst_hbm.at[pl.ds(base, _BSZ)],
                                  didx[b], isem[b]).wait()
        plsc.subcore_barrier()
        pltpu.sync_copy(acc.at[pl.ds(s * rpt, rpt)],
                        out_hbm.at[pl.ds(c * np_pad + s * rpt, rpt)])

    return k(dst)


def _agg_partials(tbl, src, dst, np_pad, npt, d, nbuf, tc_tiling):
    rpt = np_pad // _NS
    zr = 16
    nring = 2 * nbuf
    mesh = plsc.VectorSubcoreMesh(core_axis_name="c", subcore_axis_name="s")

    @functools.partial(
        pl.kernel,
        out_type=jax.ShapeDtypeStruct((_NC * np_pad, d), jnp.float32),
        mesh=mesh,
        compiler_params=pltpu.CompilerParams(use_tc_tiling_on_sc=tc_tiling),
        scratch_types=[
            [pltpu.VMEM((_BSZ,), jnp.int32) for _ in range(nring)],
            [pltpu.VMEM((_BSZ,), jnp.int32) for _ in range(nring)],
            [pltpu.SemaphoreType.DMA for _ in range(nring)],
            [pltpu.VMEM((_BSZ, d), jnp.float32) for _ in range(nbuf)],
            [pltpu.SemaphoreType.DMA for _ in range(nbuf)],
            pltpu.VMEM((zr, d), jnp.float32),
            pltpu.VMEM_SHARED((np_pad, d), jnp.float32),
        ],
    )
    def k(tbl_hbm, src_hbm, dst_hbm, out_hbm,
          sidx, didx, isem, rows, gsem, zblk, acc):
        c = lax.axis_index("c")
        s = lax.axis_index("s")
        wid = c * _NS + s
        base = wid * npt * _BSZ

        def fetch_idx(slot, j):
            off = base + j * _BSZ
            pltpu.async_copy(src_hbm.at[pl.ds(off, _BSZ)], sidx[slot],
                             isem[slot])
            pltpu.async_copy(dst_hbm.at[pl.ds(off, _BSZ)], didx[slot],
                             isem[slot])

        def wait_idx(slot):
            pltpu.make_async_copy(src_hbm.at[pl.ds(base, _BSZ)], sidx[slot],
                                  isem[slot]).wait()
            pltpu.make_async_copy(dst_hbm.at[pl.ds(base, _BSZ)], didx[slot],
                                  isem[slot]).wait()

        for b in range(nring):
            fetch_idx(b, b)
        for b in range(nbuf):
            wait_idx(b)
            pltpu.async_copy(tbl_hbm.at[sidx[b]], rows[b], gsem[b])
        for r in range(zr):
            for j in range(d // _L):
                zblk[r, pl.ds(j * _L, _L)] = jnp.zeros((_L,), jnp.float32)
        for t in range(rpt // zr):
            pltpu.sync_copy(zblk, acc.at[pl.ds(s * rpt + t * zr, zr)])
        plsc.subcore_barrier()

        def body(g, carry):
            for u in range(nring):
                br = u % nbuf
                bi2 = (u + nbuf) % nring
                j = g * nring + u
                pltpu.make_async_copy(
                    tbl_hbm.at[sidx[u]], rows[br], gsem[br]).wait()
                pltpu.sync_copy(rows[br], acc.at[didx[u]], add=True)
                fetch_idx(u, j + nring)
                wait_idx(bi2)
                pltpu.async_copy(tbl_hbm.at[sidx[bi2]], rows[br], gsem[br])
            return carry

        lax.fori_loop(0, npt // nring, body, 0)
        for b in range(nbuf):
            pltpu.make_async_copy(
                tbl_hbm.at[sidx[b]], rows[b], gsem[b]).wait()
        for b in range(nbuf, nring):
            wait_idx(b)
        plsc.subcore_barrier()
        pltpu.sync_copy(acc.at[pl.ds(s * rpt, rpt)],
                        out_hbm.at[pl.ds(c * np_pad + s * rpt, rpt)])

    return k(tbl, src, dst)


def _tc_matmul(x_pad, w1):
    np_pad, d = x_pad.shape
    blk = 1280
    grid = np_pad // blk

    def body(x_ref, w_ref, o_ref):
        o_ref[...] = jnp.dot(x_ref[...], w_ref[...],
                             preferred_element_type=jnp.float32)

    return pl.pallas_call(
        body,
        grid=(grid,),
        in_specs=[
            pl.BlockSpec((blk, d), lambda i: (i, 0)),
            pl.BlockSpec((d, d), lambda i: (0, 0)),
        ],
        out_specs=pl.BlockSpec((blk, d), lambda i: (i, 0)),
        out_shape=jax.ShapeDtypeStruct((np_pad, d), jnp.float32),
    )(x_pad, w1)


def _tc_scale(degp_t, u1):
    np_pad, d = u1.shape
    blk = 2048
    grid = np_pad // blk

    def body(deg_ref, u_ref, o_ref):
        dval = deg_ref[:, 0:1] + deg_ref[:, 1:2] + 1.0
        o_ref[...] = u_ref[...] * lax.rsqrt(dval)

    return pl.pallas_call(
        body,
        grid=(grid,),
        in_specs=[
            pl.BlockSpec((blk, 2), lambda i: (i, 0)),
            pl.BlockSpec((blk, d), lambda i: (i, 0)),
        ],
        out_specs=pl.BlockSpec((blk, d), lambda i: (i, 0)),
        out_shape=jax.ShapeDtypeStruct((np_pad, d), jnp.float32),
    )(degp_t, u1)


def _tc_layer2(p, t1, degp_t, b1, wcat):
    np_pad, d = t1.shape
    d2 = wcat.shape[1]
    blk = 1280
    grid = np_pad // blk

    def body(p_ref, t1_ref, deg_ref, b_ref, w_ref, o_ref):
        dval = deg_ref[:, 0:1] + deg_ref[:, 1:2] + 1.0
        dis = lax.rsqrt(dval)
        agg = p_ref[0] + p_ref[1] + t1_ref[...]
        h = jnp.maximum(agg * dis + b_ref[...], 0.0)
        o_ref[...] = jnp.dot(h, w_ref[...],
                             preferred_element_type=jnp.float32) * dis

    return pl.pallas_call(
        body,
        grid=(grid,),
        in_specs=[
            pl.BlockSpec((2, blk, d), lambda i: (0, i, 0)),
            pl.BlockSpec((blk, d), lambda i: (i, 0)),
            pl.BlockSpec((blk, 2), lambda i: (i, 0)),
            pl.BlockSpec((1, d), lambda i: (0, 0)),
            pl.BlockSpec((d, d2), lambda i: (0, 0)),
        ],
        out_specs=pl.BlockSpec((blk, d2), lambda i: (i, 0)),
        out_shape=jax.ShapeDtypeStruct((np_pad, d2), jnp.float32),
    )(p, t1, degp_t, b1, wcat)


def _tc_final(q, t2, degp_t, bcat):
    np_pad, d2 = t2.shape
    blk = 1280
    grid = np_pad // blk

    def body(q_ref, t2_ref, deg_ref, b_ref, o_ref):
        dval = deg_ref[:, 0:1] + deg_ref[:, 1:2] + 1.0
        dis = lax.rsqrt(dval)
        o_ref[...] = (q_ref[0] + q_ref[1] + t2_ref[...]) * dis + b_ref[...]

    return pl.pallas_call(
        body,
        grid=(grid,),
        in_specs=[
            pl.BlockSpec((2, blk, d2), lambda i: (0, i, 0)),
            pl.BlockSpec((blk, d2), lambda i: (i, 0)),
            pl.BlockSpec((blk, 2), lambda i: (i, 0)),
            pl.BlockSpec((1, d2), lambda i: (0, 0)),
        ],
        out_specs=pl.BlockSpec((blk, d2), lambda i: (i, 0)),
        out_shape=jax.ShapeDtypeStruct((np_pad, d2), jnp.float32),
    )(q, t2, degp_t, bcat)


def kernel(x, edge_index, W1, b1, W2, b2, Wd, bd):
    n, d = x.shape
    e = edge_index.shape[1]
    nc = W2.shape[1]
    nd = Wd.shape[1]
    d2 = 48

    e_pad = -(-e // (_NW * _BSZ * 8)) * (_NW * _BSZ * 8)
    npt = e_pad // (_BSZ * _NW)
    pad_s = n + (jnp.arange(e_pad + 8 * _BSZ - e, dtype=edge_index.dtype)
                 % (_NP - n))
    src_p = jnp.concatenate([edge_index[0], pad_s])
    dst_p = jnp.concatenate([edge_index[1], pad_s])

    x_pad = jnp.zeros((_NP, d), x.dtype).at[:n].set(x)
    wcat = jnp.zeros((d, d2), W2.dtype).at[:, :nc].set(W2).at[:, nc:nc + nd].set(Wd)
    bcat = jnp.zeros((1, d2), b2.dtype).at[0, :nc].set(b2).at[0, nc:nc + nd].set(bd)

    u1 = _tc_matmul(x_pad, W1)
    degp_t = _deg_partials(dst_p, _NP, npt).reshape(_NC, _NP).T
    t1 = _tc_scale(degp_t, u1)
    p = _agg_partials(t1, src_p, dst_p, _NP, npt, d,
                      nbuf=2, tc_tiling=True).reshape(_NC, _NP, d)
    t2 = _tc_layer2(p, t1, degp_t, b1.reshape(1, d), wcat)
    q = _agg_partials(t2, src_p, dst_p, _NP, npt, d2,
                      nbuf=4, tc_tiling=False).reshape(_NC, _NP, d2)
    out = _tc_final(q, t2, degp_t, bcat)
    return out[:n, :nc], out[:n, nc:nc + nd]


# speedup vs baseline: 1.0182x; 1.0182x over previous
import functools

import jax
import jax.numpy as jnp
from jax import lax
from jax.experimental import pallas as pl
from jax.experimental.pallas import tpu as pltpu
from jax.experimental.pallas import tpu_sc as plsc

_NC, _NS, _L = 2, 16, 16
_NW = _NC * _NS
_NP = 10240
_BSZ = 128


def _deg_partials(dst, np_pad, npt):
    rpt = np_pad // _NS
    nring = 4
    mesh = plsc.VectorSubcoreMesh(core_axis_name="c", subcore_axis_name="s")

    @functools.partial(
        pl.kernel,
        out_type=jax.ShapeDtypeStruct((_NC * np_pad,), jnp.float32),
        mesh=mesh,
        scratch_types=[
            [pltpu.VMEM((_BSZ,), jnp.int32) for _ in range(nring)],
            [pltpu.SemaphoreType.DMA for _ in range(nring)],
            pltpu.VMEM((_BSZ,), jnp.float32),
            pltpu.VMEM((rpt,), jnp.float32),
            pltpu.VMEM_SHARED((np_pad,), jnp.float32),
        ],
    )
    def k(dst_hbm, out_hbm, didx, isem, ones_v, zeros_v, acc):
        c = lax.axis_index("c")
        s = lax.axis_index("s")
        wid = c * _NS + s
        base = wid * npt * _BSZ
        for b in range(nring):
            pltpu.async_copy(dst_hbm.at[pl.ds(base + b * _BSZ, _BSZ)],
                             didx[b], isem[b])
        for j in range(_BSZ // _L):
            ones_v[pl.ds(j * _L, _L)] = jnp.full((_L,), 1.0, jnp.float32)
        for j in range(rpt // _L):
            zeros_v[pl.ds(j * _L, _L)] = jnp.zeros((_L,), jnp.float32)
        pltpu.sync_copy(zeros_v, acc.at[pl.ds(s * rpt, rpt)])
        plsc.subcore_barrier()

        def body(g, carry):
            for u in range(nring):
                j = g * nring + u
                pltpu.make_async_copy(dst_hbm.at[pl.ds(base, _BSZ)],
                                      didx[u], isem[u]).wait()
                pltpu.sync_copy(ones_v, acc.at[didx[u]], add=True)
                pltpu.async_copy(
                    dst_hbm.at[pl.ds(base + (j + nring) * _BSZ, _BSZ)],
                    didx[u], isem[u])
            return carry

        lax.fori_loop(0, npt // nring, body, 0)
        for b in range(nring):
            pltpu.make_async_copy(dst_hbm.at[pl.ds(base, _BSZ)],
                                  didx[b], isem[b]).wait()
        plsc.subcore_barrier()
        pltpu.sync_copy(acc.at[pl.ds(s * rpt, rpt)],
                        out_hbm.at[pl.ds(c * np_pad + s * rpt, rpt)])

    return k(dst)


def _agg_partials(tbl, src, dst, np_pad, npt, d, nbuf, tc_tiling):
    rpt = np_pad // _NS
    zr = 16
    nring = 2 * nbuf
    mesh = plsc.VectorSubcoreMesh(core_axis_name="c", subcore_axis_name="s")

    @functools.partial(
        pl.kernel,
        out_type=jax.ShapeDtypeStruct((_NC * np_pad, d), jnp.float32),
        mesh=mesh,
        compiler_params=pltpu.CompilerParams(use_tc_tiling_on_sc=tc_tiling),
        scratch_types=[
            [pltpu.VMEM((_BSZ,), jnp.int32) for _ in range(nring)],
            [pltpu.VMEM((_BSZ,), jnp.int32) for _ in range(nring)],
            [pltpu.SemaphoreType.DMA for _ in range(nring)],
            [pltpu.VMEM((_BSZ, d), jnp.float32) for _ in range(nbuf)],
            [pltpu.SemaphoreType.DMA for _ in range(nbuf)],
            pltpu.VMEM((zr, d), jnp.float32),
            pltpu.VMEM_SHARED((np_pad, d), jnp.float32),
        ],
    )
    def k(tbl_hbm, src_hbm, dst_hbm, out_hbm,
          sidx, didx, isem, rows, gsem, zblk, acc):
        c = lax.axis_index("c")
        s = lax.axis_index("s")
        wid = c * _NS + s
        base = wid * npt * _BSZ

        def fetch_idx(slot, j):
            off = base + j * _BSZ
            pltpu.async_copy(src_hbm.at[pl.ds(off, _BSZ)], sidx[slot],
                             isem[slot])
            pltpu.async_copy(dst_hbm.at[pl.ds(off, _BSZ)], didx[slot],
                             isem[slot])

        def wait_idx(slot):
            pltpu.make_async_copy(src_hbm.at[pl.ds(base, _BSZ)], sidx[slot],
                                  isem[slot]).wait()
            pltpu.make_async_copy(dst_hbm.at[pl.ds(base, _BSZ)], didx[slot],
                                  isem[slot]).wait()

        for b in range(nring):
            fetch_idx(b, b)
        for b in range(nbuf):
            wait_idx(b)
            pltpu.async_copy(tbl_hbm.at[sidx[b]], rows[b], gsem[b])
        for r in range(zr):
            for j in range(d // _L):
                zblk[r, pl.ds(j * _L, _L)] = jnp.zeros((_L,), jnp.float32)
        for t in range(rpt // zr):
            pltpu.sync_copy(zblk, acc.at[pl.ds(s * rpt + t * zr, zr)])
        plsc.subcore_barrier()

        def body(g, carry):
            for u in range(nring):
                br = u % nbuf
                bi2 = (u + nbuf) % nring
                j = g * nring + u
                pltpu.make_async_copy(
                    tbl_hbm.at[sidx[u]], rows[br], gsem[br]).wait()
                pltpu.sync_copy(rows[br], acc.at[didx[u]], add=True)
                fetch_idx(u, j + nring)
                wait_idx(bi2)
                pltpu.async_copy(tbl_hbm.at[sidx[bi2]], rows[br], gsem[br])
            return carry

        lax.fori_loop(0, npt // nring, body, 0)
        for b in range(nbuf):
            pltpu.make_async_copy(
                tbl_hbm.at[sidx[b]], rows[b], gsem[b]).wait()
        for b in range(nbuf, nring):
            wait_idx(b)
        plsc.subcore_barrier()
        pltpu.sync_copy(acc.at[pl.ds(s * rpt, rpt)],
                        out_hbm.at[pl.ds(c * np_pad + s * rpt, rpt)])

    return k(tbl, src, dst)


def _tc_prep1(degp_t, x_pad, w1):
    np_pad, d = x_pad.shape
    blk = 2048
    grid = np_pad // blk

    def body(deg_ref, x_ref, w_ref, o_ref):
        dval = deg_ref[:, 0:1] + deg_ref[:, 1:2] + 1.0
        dis = lax.rsqrt(dval)
        o_ref[...] = jnp.dot(x_ref[...], w_ref[...],
                             preferred_element_type=jnp.float32) * dis

    return pl.pallas_call(
        body,
        grid=(grid,),
        in_specs=[
            pl.BlockSpec((blk, 2), lambda i: (i, 0)),
            pl.BlockSpec((blk, d), lambda i: (i, 0)),
            pl.BlockSpec((d, d), lambda i: (0, 0)),
        ],
        out_specs=pl.BlockSpec((blk, d), lambda i: (i, 0)),
        out_shape=jax.ShapeDtypeStruct((np_pad, d), jnp.float32),
    )(degp_t, x_pad, w1)


def _tc_layer2(p, t1, degp_t, b1, wcat):
    np_pad, d = t1.shape
    d2 = wcat.shape[1]
    blk = 2048
    grid = np_pad // blk

    def body(p_ref, t1_ref, deg_ref, b_ref, w_ref, o_ref):
        dval = deg_ref[:, 0:1] + deg_ref[:, 1:2] + 1.0
        dis = lax.rsqrt(dval)
        agg = p_ref[0] + p_ref[1] + t1_ref[...]
        h = jnp.maximum(agg * dis + b_ref[...], 0.0)
        o_ref[...] = jnp.dot(h, w_ref[...],
                             preferred_element_type=jnp.float32) * dis

    return pl.pallas_call(
        body,
        grid=(grid,),
        in_specs=[
            pl.BlockSpec((2, blk, d), lambda i: (0, i, 0)),
            pl.BlockSpec((blk, d), lambda i: (i, 0)),
            pl.BlockSpec((blk, 2), lambda i: (i, 0)),
            pl.BlockSpec((1, d), lambda i: (0, 0)),
            pl.BlockSpec((d, d2), lambda i: (0, 0)),
        ],
        out_specs=pl.BlockSpec((blk, d2), lambda i: (i, 0)),
        out_shape=jax.ShapeDtypeStruct((np_pad, d2), jnp.float32),
    )(p, t1, degp_t, b1, wcat)


def _tc_final(q, t2, degp_t, bcat):
    np_pad, d2 = t2.shape
    blk = 2048
    grid = np_pad // blk

    def body(q_ref, t2_ref, deg_ref, b_ref, o_ref):
        dval = deg_ref[:, 0:1] + deg_ref[:, 1:2] + 1.0
        dis = lax.rsqrt(dval)
        o_ref[...] = (q_ref[0] + q_ref[1] + t2_ref[...]) * dis + b_ref[...]

    return pl.pallas_call(
        body,
        grid=(grid,),
        in_specs=[
            pl.BlockSpec((2, blk, d2), lambda i: (0, i, 0)),
            pl.BlockSpec((blk, d2), lambda i: (i, 0)),
            pl.BlockSpec((blk, 2), lambda i: (i, 0)),
            pl.BlockSpec((1, d2), lambda i: (0, 0)),
        ],
        out_specs=pl.BlockSpec((blk, d2), lambda i: (i, 0)),
        out_shape=jax.ShapeDtypeStruct((np_pad, d2), jnp.float32),
    )(q, t2, degp_t, bcat)


def kernel(x, edge_index, W1, b1, W2, b2, Wd, bd):
    n, d = x.shape
    e = edge_index.shape[1]
    nc = W2.shape[1]
    nd = Wd.shape[1]
    d2 = 48

    e_pad = -(-e // (_NW * _BSZ * 8)) * (_NW * _BSZ * 8)
    npt = e_pad // (_BSZ * _NW)
    pad_s = n + (jnp.arange(e_pad + 8 * _BSZ - e, dtype=edge_index.dtype)
                 % (_NP - n))
    src_p = jnp.concatenate([edge_index[0], pad_s])
    dst_p = jnp.concatenate([edge_index[1], pad_s])

    x_pad = jnp.zeros((_NP, d), x.dtype).at[:n].set(x)
    wcat = jnp.zeros((d, d2), W2.dtype).at[:, :nc].set(W2).at[:, nc:nc + nd].set(Wd)
    bcat = jnp.zeros((1, d2), b2.dtype).at[0, :nc].set(b2).at[0, nc:nc + nd].set(bd)

    degp_t = _deg_partials(dst_p, _NP, npt).reshape(_NC, _NP).T
    t1 = _tc_prep1(degp_t, x_pad, W1)
    p = _agg_partials(t1, src_p, dst_p, _NP, npt, d,
                      nbuf=2, tc_tiling=True).reshape(_NC, _NP, d)
    t2 = _tc_layer2(p, t1, degp_t, b1.reshape(1, d), wcat)
    q = _agg_partials(t2, src_p, dst_p, _NP, npt, d2,
                      nbuf=4, tc_tiling=False).reshape(_NC, _NP, d2)
    out = _tc_final(q, t2, degp_t, bcat)
    return out[:n, :nc], out[:n, nc:nc + nd]


# speedup vs baseline: 1.0203x; 1.0021x over previous
import functools

import jax
import jax.numpy as jnp
from jax import lax
from jax.experimental import pallas as pl
from jax.experimental.pallas import tpu as pltpu
from jax.experimental.pallas import tpu_sc as plsc

_NC, _NS, _L = 2, 16, 16
_NW = _NC * _NS
_NP = 10240
_BSZ = 128


def _deg_partials(dst, np_pad, npt):
    rpt = np_pad // _NS
    nring = 4
    mesh = plsc.VectorSubcoreMesh(core_axis_name="c", subcore_axis_name="s")

    @functools.partial(
        pl.kernel,
        out_type=jax.ShapeDtypeStruct((_NC * np_pad,), jnp.float32),
        mesh=mesh,
        scratch_types=[
            [pltpu.VMEM((_BSZ,), jnp.int32) for _ in range(nring)],
            [pltpu.SemaphoreType.DMA for _ in range(nring)],
            pltpu.VMEM((_BSZ,), jnp.float32),
            pltpu.VMEM((rpt,), jnp.float32),
            pltpu.VMEM_SHARED((np_pad,), jnp.float32),
        ],
    )
    def k(dst_hbm, out_hbm, didx, isem, ones_v, zeros_v, acc):
        c = lax.axis_index("c")
        s = lax.axis_index("s")
        wid = c * _NS + s
        base = wid * npt * _BSZ
        for b in range(nring):
            pltpu.async_copy(dst_hbm.at[pl.ds(base + b * _BSZ, _BSZ)],
                             didx[b], isem[b])
        for j in range(_BSZ // _L):
            ones_v[pl.ds(j * _L, _L)] = jnp.full((_L,), 1.0, jnp.float32)
        for j in range(rpt // _L):
            zeros_v[pl.ds(j * _L, _L)] = jnp.zeros((_L,), jnp.float32)
        pltpu.sync_copy(zeros_v, acc.at[pl.ds(s * rpt, rpt)])
        plsc.subcore_barrier()

        def body(g, carry):
            for u in range(nring):
                j = g * nring + u
                pltpu.make_async_copy(dst_hbm.at[pl.ds(base, _BSZ)],
                                      didx[u], isem[u]).wait()
                pltpu.sync_copy(ones_v, acc.at[didx[u]], add=True)
                pltpu.async_copy(
                    dst_hbm.at[pl.ds(base + (j + nring) * _BSZ, _BSZ)],
                    didx[u], isem[u])
            return carry

        lax.fori_loop(0, npt // nring, body, 0)
        for b in range(nring):
            pltpu.make_async_copy(dst_hbm.at[pl.ds(base, _BSZ)],
                                  didx[b], isem[b]).wait()
        plsc.subcore_barrier()
        pltpu.sync_copy(acc.at[pl.ds(s * rpt, rpt)],
                        out_hbm.at[pl.ds(c * np_pad + s * rpt, rpt)])

    return k(dst)


def _agg_partials(tbl, src, dst, np_pad, npt, d, nbuf, tc_tiling):
    rpt = np_pad // _NS
    zr = 16
    nring = 2 * nbuf
    mesh = plsc.VectorSubcoreMesh(core_axis_name="c", subcore_axis_name="s")

    @functools.partial(
        pl.kernel,
        out_type=jax.ShapeDtypeStruct((_NC * np_pad, d), jnp.float32),
        mesh=mesh,
        compiler_params=pltpu.CompilerParams(use_tc_tiling_on_sc=tc_tiling),
        scratch_types=[
            [pltpu.VMEM((_BSZ,), jnp.int32) for _ in range(nring)],
            [pltpu.VMEM((_BSZ,), jnp.int32) for _ in range(nring)],
            [pltpu.SemaphoreType.DMA for _ in range(nring)],
            [pltpu.VMEM((_BSZ, d), jnp.float32) for _ in range(nbuf)],
            [pltpu.SemaphoreType.DMA for _ in range(nbuf)],
            pltpu.VMEM((zr, d), jnp.float32),
            pltpu.VMEM_SHARED((np_pad, d), jnp.float32),
        ],
    )
    def k(tbl_hbm, src_hbm, dst_hbm, out_hbm,
          sidx, didx, isem, rows, gsem, zblk, acc):
        c = lax.axis_index("c")
        s = lax.axis_index("s")
        wid = c * _NS + s
        base = wid * npt * _BSZ

        def fetch_idx(slot, j):
            off = base + j * _BSZ
            pltpu.async_copy(src_hbm.at[pl.ds(off, _BSZ)], sidx[slot],
                             isem[slot])
            pltpu.async_copy(dst_hbm.at[pl.ds(off, _BSZ)], didx[slot],
                             isem[slot])

        def wait_idx(slot):
            pltpu.make_async_copy(src_hbm.at[pl.ds(base, _BSZ)], sidx[slot],
                                  isem[slot]).wait()
            pltpu.make_async_copy(dst_hbm.at[pl.ds(base, _BSZ)], didx[slot],
                                  isem[slot]).wait()

        for b in range(nring):
            fetch_idx(b, b)
        for b in range(nbuf):
            wait_idx(b)
            pltpu.async_copy(tbl_hbm.at[sidx[b]], rows[b], gsem[b])
        for r in range(zr):
            for j in range(d // _L):
                zblk[r, pl.ds(j * _L, _L)] = jnp.zeros((_L,), jnp.float32)
        for t in range(rpt // zr):
            pltpu.sync_copy(zblk, acc.at[pl.ds(s * rpt + t * zr, zr)])
        plsc.subcore_barrier()

        def body(g, carry):
            for u in range(nring):
                br = u % nbuf
                bi2 = (u + nbuf) % nring
                j = g * nring + u
                pltpu.make_async_copy(
                    tbl_hbm.at[sidx[u]], rows[br], gsem[br]).wait()
                pltpu.sync_copy(rows[br], acc.at[didx[u]], add=True)
                fetch_idx(u, j + nring)
                wait_idx(bi2)
                pltpu.async_copy(tbl_hbm.at[sidx[bi2]], rows[br], gsem[br])
            return carry

        lax.fori_loop(0, npt // nring, body, 0)
        for b in range(nbuf):
            pltpu.make_async_copy(
                tbl_hbm.at[sidx[b]], rows[b], gsem[b]).wait()
        for b in range(nbuf, nring):
            wait_idx(b)
        plsc.subcore_barrier()
        pltpu.sync_copy(acc.at[pl.ds(s * rpt, rpt)],
                        out_hbm.at[pl.ds(c * np_pad + s * rpt, rpt)])

    return k(tbl, src, dst)


def _tc_prep1(degp_t, x_pad, w1):
    np_pad, d = x_pad.shape
    blk = 2048
    grid = np_pad // blk

    def body(deg_ref, x_ref, w_ref, o_ref):
        dval = deg_ref[:, 0:1] + deg_ref[:, 1:2] + 1.0
        dis = lax.rsqrt(dval)
        o_ref[...] = jnp.dot(x_ref[...], w_ref[...],
                             preferred_element_type=jnp.float32) * dis

    return pl.pallas_call(
        body,
        grid=(grid,),
        in_specs=[
            pl.BlockSpec((blk, 2), lambda i: (i, 0)),
            pl.BlockSpec((blk, d), lambda i: (i, 0)),
            pl.BlockSpec((d, d), lambda i: (0, 0)),
        ],
        out_specs=pl.BlockSpec((blk, d), lambda i: (i, 0)),
        out_shape=jax.ShapeDtypeStruct((np_pad, d), jnp.float32),
    )(degp_t, x_pad, w1)


def _tc_layer2(p, t1, degp_t, b1, wcat):
    np_pad, d = t1.shape
    d2 = wcat.shape[1]
    blk = 2048
    grid = np_pad // blk

    def body(p_ref, t1_ref, deg_ref, b_ref, w_ref, o_ref):
        dval = deg_ref[:, 0:1] + deg_ref[:, 1:2] + 1.0
        dis = lax.rsqrt(dval)
        agg = p_ref[0] + p_ref[1] + t1_ref[...]
        h = jnp.maximum(agg * dis + b_ref[...], 0.0)
        o_ref[...] = jnp.dot(h, w_ref[...],
                             preferred_element_type=jnp.float32) * dis

    return pl.pallas_call(
        body,
        grid=(grid,),
        in_specs=[
            pl.BlockSpec((2, blk, d), lambda i: (0, i, 0)),
            pl.BlockSpec((blk, d), lambda i: (i, 0)),
            pl.BlockSpec((blk, 2), lambda i: (i, 0)),
            pl.BlockSpec((1, d), lambda i: (0, 0)),
            pl.BlockSpec((d, d2), lambda i: (0, 0)),
        ],
        out_specs=pl.BlockSpec((blk, d2), lambda i: (i, 0)),
        out_shape=jax.ShapeDtypeStruct((np_pad, d2), jnp.float32),
    )(p, t1, degp_t, b1, wcat)


def _tc_final(q, t2, degp_t, bcat):
    np_pad, d2 = t2.shape
    blk = 2048
    grid = np_pad // blk

    def body(q_ref, t2_ref, deg_ref, b_ref, o_ref):
        dval = deg_ref[:, 0:1] + deg_ref[:, 1:2] + 1.0
        dis = lax.rsqrt(dval)
        o_ref[...] = (q_ref[0] + q_ref[1] + t2_ref[...]) * dis + b_ref[...]

    return pl.pallas_call(
        body,
        grid=(grid,),
        in_specs=[
            pl.BlockSpec((2, blk, d2), lambda i: (0, i, 0)),
            pl.BlockSpec((blk, d2), lambda i: (i, 0)),
            pl.BlockSpec((blk, 2), lambda i: (i, 0)),
            pl.BlockSpec((1, d2), lambda i: (0, 0)),
        ],
        out_specs=pl.BlockSpec((blk, d2), lambda i: (i, 0)),
        out_shape=jax.ShapeDtypeStruct((np_pad, d2), jnp.float32),
    )(q, t2, degp_t, bcat)


def kernel(x, edge_index, W1, b1, W2, b2, Wd, bd):
    n, d = x.shape
    e = edge_index.shape[1]
    nc = W2.shape[1]
    nd = Wd.shape[1]
    d2 = 48

    e_pad = -(-e // (_NW * _BSZ * 8)) * (_NW * _BSZ * 8)
    npt = e_pad // (_BSZ * _NW)
    pad_s = n + (jnp.arange(e_pad + 8 * _BSZ - e, dtype=edge_index.dtype)
                 % (_NP - n))
    src_p = jnp.concatenate([edge_index[0], pad_s])
    dst_p = jnp.concatenate([edge_index[1], pad_s])

    x_pad = jnp.zeros((_NP, d), x.dtype).at[:n].set(x)
    wcat = jnp.zeros((d, d2), W2.dtype).at[:, :nc].set(W2).at[:, nc:nc + nd].set(Wd)
    bcat = jnp.zeros((1, d2), b2.dtype).at[0, :nc].set(b2).at[0, nc:nc + nd].set(bd)

    degp_t = _deg_partials(dst_p, _NP, npt).reshape(_NC, _NP).T
    t1 = _tc_prep1(degp_t, x_pad, W1)
    p = _agg_partials(t1, src_p, dst_p, _NP, npt, d,
                      nbuf=2, tc_tiling=False).reshape(_NC, _NP, d)
    t2 = _tc_layer2(p, t1, degp_t, b1.reshape(1, d), wcat)
    q = _agg_partials(t2, src_p, dst_p, _NP, npt, d2,
                      nbuf=4, tc_tiling=False).reshape(_NC, _NP, d2)
    out = _tc_final(q, t2, degp_t, bcat)
    return out[:n, :nc], out[:n, nc:nc + nd]
